# baseline (speedup 1.0000x reference)
import jax
import jax.numpy as jnp
from jax.experimental import pallas as pl

_ROWS_PER_BLOCK = 2000


def _proj_kernel(x_ref, w_ref, b_ref, o_ref):
    o_ref[...] = (
        jnp.dot(x_ref[...], w_ref[...], preferred_element_type=jnp.float32)
        + b_ref[...]
    )


def _project(movie_x, W, b):
    m, k = movie_x.shape
    n = W.shape[1]
    grid = (m // _ROWS_PER_BLOCK,) if m % _ROWS_PER_BLOCK == 0 else (
        pl.cdiv(m, _ROWS_PER_BLOCK),)
    return pl.pallas_call(
        _proj_kernel,
        grid=grid,
        in_specs=[
            pl.BlockSpec((_ROWS_PER_BLOCK, k), lambda i: (i, 0)),
            pl.BlockSpec((k, n), lambda i: (0, 0)),
            pl.BlockSpec((n,), lambda i: (0,)),
        ],
        out_specs=pl.BlockSpec((_ROWS_PER_BLOCK, n), lambda i: (i, 0)),
        out_shape=jax.ShapeDtypeStruct((m, n), jnp.float32),
    )(movie_x, W, b)


def kernel(movie_x, user_emb_weight, W, b):
    movie = _project(movie_x, W, b)
    return (user_emb_weight, movie)
